# baseline (speedup 1.0000x reference)
import functools

import jax
import jax.numpy as jnp
from jax import lax
from jax.experimental import pallas as pl
from jax.experimental.pallas import tpu as pltpu
from jax.experimental.pallas import tpu_sc as plsc

_N = 256
_D = 128
_MARGIN = 1.0
_BIG = 1e9
_NW = 32
_APT = _N // _NW
_L = 16
_NCH = _N // _L


def _dist_kernel(emb_ref, d_ref):
    E = emb_ref[:]
    G = lax.dot_general(E, E, (((1,), (1,)), ((), ())),
                        preferred_element_type=jnp.float32)
    EE = E * E
    sq_col = jnp.sum(EE, axis=1, keepdims=True)
    ones_d = jnp.ones((1, _D), jnp.float32)
    sq_row = lax.dot_general(ones_d, EE, (((1,), (1,)), ((), ())),
                             preferred_element_type=jnp.float32)
    d_ref[...] = sq_col + sq_row - 2.0 * G


def _sc_body(d_hbm, t_hbm, out_hbm, t_v, d_v, b_v, av_v, fold_v, wl_v,
             macc_v, out_v):
    wid = lax.axis_index("s") * 2 + lax.axis_index("c")
    base = wid * _APT

    pltpu.sync_copy(t_hbm, t_v.at[pl.ds(0, _N)])
    pltpu.sync_copy(d_hbm.at[pl.ds(base * _N, _APT * _N)], d_v)

    iota = lax.iota(jnp.int32, _L)
    zf = jnp.zeros((_L,), jnp.float32)
    negbig = zf - _BIG
    lt = {sh: iota < (_L - sh) for sh in (8, 4, 2, 1)}
    tot_acc, vio_acc, cnt_acc = zf, zf, zf

    gn = jnp.int32(0)
    for a in range(_APT):
        i = base + a
        t_i = t_v[pl.ds(i, _L)][0]
        macc = zf
        wl = jnp.zeros((_L,), jnp.int32)
        nact = jnp.int32(0)
        for ch in range(_NCH):
            cid = a * _NCH + ch
            choff = ch * _L
            kidx = choff + iota
            tch = t_v[pl.ds(choff, _L)]
            dch = d_v[pl.ds(a * _N + choff, _L)]
            same = tch == t_i
            b_v[pl.ds(a * _N + choff, _L)] = jnp.where(same, _BIG, dch)
            macc = macc + jnp.where(same, 0.0, 1.0)
            ap = same & (kidx > i)
            av = jnp.where(ap, dch + _MARGIN, -_BIG)
            av_v[pl.ds(cid * _L, _L)] = av
            cur = av
            for sh in (8, 4, 2, 1):
                fold_v[pl.ds(cid * _L, _L)] = cur
                w = fold_v[pl.ds(cid * _L + sh, _L)]
                cur = jnp.maximum(cur, jnp.where(lt[sh], w, negbig))
            hit = cur[0] > 0.0
            wl = jnp.where(iota == nact, cid, wl)
            nact = nact + jnp.where(hit, 1, 0)
        macc_v[pl.ds(a * _L, _L)] = macc
        wl_v[pl.ds(gn, _L)] = wl
        gn = gn + nact

    def do_chunk(q, carry):
        tot0, tot1, vio0, vio1, cnt = carry
        cid = wl_v[pl.ds(q, _L)][0]
        a_q = lax.shift_right_logical(cid, 4)
        boff = a_q * _N
        av = av_v[pl.ds(cid * _L, _L)]
        avs = [av[l] for l in range(_L)]
        macc = macc_v[pl.ds(a_q * _L, _L)]
        accs = [tot0, tot1, vio0, vio1]
        for ch2 in range(_NCH):
            bch = b_v[pl.ds(boff + ch2 * _L, _L)]
            for l in range(_L):
                t = avs[l] - bch
                s = (ch2 + l) % 2
                accs[s] = accs[s] + jnp.maximum(t, 0.0)
                accs[2 + s] = accs[2 + s] + jnp.where(t > 0.0, 1.0, 0.0)
        for l in range(_L):
            cnt = cnt + jnp.where(avs[l] > 0.0, macc, zf)
        return accs[0], accs[1], accs[2], accs[3], cnt

    tot0, tot1, vio0, vio1, cnt_acc = lax.fori_loop(
        0, gn, do_chunk, (zf, zf, zf, zf, cnt_acc))
    tot_acc = tot0 + tot1
    vio_acc = vio0 + vio1

    out_v[pl.ds(0, _L)] = tot_acc
    out_v[pl.ds(_L, _L)] = vio_acc
    out_v[pl.ds(2 * _L, _L)] = cnt_acc
    out_v[pl.ds(3 * _L, _L)] = zf
    pltpu.sync_copy(out_v, out_hbm.at[wid])


def _sc_reduce(d_flat, t32):
    mesh = plsc.VectorSubcoreMesh(core_axis_name="c", subcore_axis_name="s")
    return pl.kernel(
        _sc_body,
        out_type=jax.ShapeDtypeStruct((_NW, 4 * _L), jnp.float32),
        mesh=mesh,
        scratch_types=[
            pltpu.VMEM((_N + _L,), jnp.int32),
            pltpu.VMEM((_APT * _N,), jnp.float32),
            pltpu.VMEM((_APT * _N,), jnp.float32),
            pltpu.VMEM((_APT * _N,), jnp.float32),
            pltpu.VMEM((_APT * _N + _L,), jnp.float32),
            pltpu.VMEM((_APT * _NCH + _L,), jnp.int32),
            pltpu.VMEM((_APT * _L,), jnp.float32),
            pltpu.VMEM((4 * _L,), jnp.float32),
        ],
    )(d_flat, t32)


def kernel(embeddings, target):
    t32 = target.astype(jnp.int32)
    dmat = pl.pallas_call(
        _dist_kernel,
        out_shape=jax.ShapeDtypeStruct((_N, _N), jnp.float32),
    )(embeddings.astype(jnp.float32))
    parts = _sc_reduce(dmat.reshape(_N * _N), t32)
    total = jnp.sum(parts[:, 0:_L])
    viol = jnp.sum(parts[:, _L:2 * _L])
    count = jnp.sum(parts[:, 2 * _L:3 * _L])
    has = count > 0.5
    loss_sum = jnp.where(has, total, jnp.float32(1.0))
    ratio = jnp.where(has, viol / jnp.maximum(count, 1.0),
                      jnp.float32(1.0))
    return (loss_sum, ratio)


# speedup vs baseline: 1.4224x; 1.4224x over previous
import jax
import jax.numpy as jnp
from jax import lax
from jax.experimental import pallas as pl
from jax.experimental.pallas import tpu as pltpu

_N = 256
_D = 128
_MARGIN = 1.0
_BIG = 1e9
_BLK = 8
_NBLK = _N // _BLK


def _triplet_kernel(emb_ref, trow_ref, tcol_ref, loss_ref, ratio_ref,
                    a_s, b_s):
    E = emb_ref[:]
    t_row = trow_ref[:]
    t_col = tcol_ref[:]

    G = lax.dot_general(E, E, (((1,), (1,)), ((), ())),
                        preferred_element_type=jnp.float32)
    EE = E * E
    sq_col = jnp.sum(EE, axis=1, keepdims=True)
    ones_d = jnp.ones((1, _D), jnp.float32)
    sq_row = lax.dot_general(ones_d, EE, (((1,), (1,)), ((), ())),
                             preferred_element_type=jnp.float32)
    Dm = sq_col + sq_row - 2.0 * G

    same = t_col == t_row
    row_i = lax.broadcasted_iota(jnp.int32, (_N, _N), 0)
    col_i = lax.broadcasted_iota(jnp.int32, (_N, _N), 1)
    apf = jnp.where(same & (row_i < col_i), 1.0, 0.0)
    negf = jnp.where(same, 0.0, 1.0)

    a_s[...] = jnp.where(apf > 0.5, Dm + _MARGIN, -_BIG)
    b_s[...] = jnp.where(negf > 0.5, Dm, _BIG)

    ones_col = jnp.ones((_N, 1), jnp.float32)
    ones_big = jnp.ones((1, _BLK * _N), jnp.float32)

    def body(bi, carry):
        tot_acc, viol_acc = carry
        i0 = bi * _BLK
        A = a_s[pl.ds(i0, _BLK), :]
        B = b_s[pl.ds(i0, _BLK), :]
        T = (A[:, :, None] - B[:, None, :]).reshape(_BLK * _N, _N)
        Cf = jnp.where(T > 0.0, 1.0, 0.0)
        R = T * Cf
        csum = lax.dot_general(Cf, ones_col, (((1,), (0,)), ((), ())),
                               preferred_element_type=jnp.float32)
        rsum = lax.dot_general(R, ones_col, (((1,), (0,)), ((), ())),
                               preferred_element_type=jnp.float32)
        vpart = lax.dot_general(ones_big, csum, (((1,), (0,)), ((), ())),
                                preferred_element_type=jnp.float32)
        tpart = lax.dot_general(ones_big, rsum, (((1,), (0,)), ((), ())),
                                preferred_element_type=jnp.float32)
        return tot_acc + tpart, viol_acc + vpart

    zeros = jnp.zeros((1, 1), jnp.float32)
    tot_acc, viol_acc = lax.fori_loop(0, _NBLK, body, (zeros, zeros))

    total = jnp.sum(tot_acc)
    viol = jnp.sum(viol_acc)

    ones_n = jnp.ones((1, _N), jnp.float32)
    p_row = lax.dot_general(ones_n, apf, (((1,), (1,)), ((), ())),
                            preferred_element_type=jnp.float32)
    m_row = lax.dot_general(ones_n, negf, (((1,), (1,)), ((), ())),
                            preferred_element_type=jnp.float32)
    count = jnp.sum(p_row * m_row)

    has = count > 0.5
    loss_sum = jnp.where(has, total, jnp.float32(1.0))
    ratio = jnp.where(has, viol / jnp.maximum(count, 1.0),
                      jnp.float32(1.0))
    loss_ref[...] = jnp.broadcast_to(loss_sum, (1, 1))
    ratio_ref[...] = jnp.broadcast_to(ratio, (1, 1))


def kernel(embeddings, target):
    t32 = target.astype(jnp.int32)
    t_row = t32.reshape(1, _N)
    t_col = t32.reshape(_N, 1)
    loss, ratio = pl.pallas_call(
        _triplet_kernel,
        out_shape=(jax.ShapeDtypeStruct((1, 1), jnp.float32),
                   jax.ShapeDtypeStruct((1, 1), jnp.float32)),
        scratch_shapes=[pltpu.VMEM((_N, _N), jnp.float32),
                        pltpu.VMEM((_N, _N), jnp.float32)],
    )(embeddings.astype(jnp.float32), t_row, t_col)
    return (loss[0, 0], ratio[0, 0])


# speedup vs baseline: 2.3640x; 1.6620x over previous
import jax
import jax.numpy as jnp
from jax import lax
from jax.experimental import pallas as pl
from jax.experimental.pallas import tpu as pltpu

_N = 256
_D = 128
_MARGIN = 1.0
_BIG = 1e9
_BLK = 8
_NBLK = _N // _BLK
_W = 64
_PAD = _N + _W * 2


def _triplet_kernel(emb_ref, trow_ref, tcol_ref, kx_ref, loss_ref,
                    ratio_ref, b_s, as3):
    E = emb_ref[:]
    t_row = trow_ref[:]
    t_col = tcol_ref[:]

    G = lax.dot_general(E, E, (((1,), (1,)), ((), ())),
                        preferred_element_type=jnp.float32)
    EE = E * E
    sq_col = jnp.sum(EE, axis=1, keepdims=True)
    ones_d = jnp.ones((1, _D), jnp.float32)
    sq_row = lax.dot_general(ones_d, EE, (((1,), (1,)), ((), ())),
                             preferred_element_type=jnp.float32)
    Dm = sq_col + sq_row - 2.0 * G

    same = t_col == t_row
    row_i = lax.broadcasted_iota(jnp.int32, (_N, _N), 0)
    col_i = lax.broadcasted_iota(jnp.int32, (_N, _N), 1)
    apf = jnp.where(same & (row_i < col_i), 1.0, 0.0)
    negf = jnp.where(same, 0.0, 1.0)

    b_s[...] = jnp.where(negf > 0.5, Dm, _BIG)

    padv = jnp.full((_PAD - _N, _BLK), -_BIG, jnp.float32)
    rowj = lax.broadcasted_iota(jnp.int32, (_N, _BLK), 0)
    cola = lax.broadcasted_iota(jnp.int32, (_N, _BLK), 1)
    for b in range(_NBLK):
        i0 = b * _BLK
        dcol = Dm[:, i0:i0 + _BLK]
        tcb = t_row[:, i0:i0 + _BLK]
        ap_cb = (t_col == tcb) & (rowj > cola + i0)
        as3[b, 0:_N, :] = jnp.where(ap_cb, dcol + _MARGIN, -_BIG)
        as3[b, _N:_PAD, :] = padv

    def body(b, carry):
        tot_acc, vio_acc = carry
        i0 = b * _BLK
        Bm = b_s[pl.ds(i0, _BLK), :]

        def win(jw, tacc, vacc):
            Ap = as3[b, pl.ds(jw, _W), :]
            T = Ap[:, :, None] - Bm[None, :, :]
            tacc = tacc + jnp.sum(jnp.maximum(T, 0.0), axis=0)
            vacc = vacc + jnp.sum(jnp.where(T > 0.0, 1.0, 0.0), axis=0)
            return tacc, vacc

        tot_acc, vio_acc = win(i0, tot_acc, vio_acc)

        def fb(q, c):
            return win(i0 + _W + q * _W, c[0], c[1])

        kx = kx_ref[0, b]
        tot_acc, vio_acc = lax.fori_loop(0, kx, fb, (tot_acc, vio_acc))
        return tot_acc, vio_acc

    zeros = jnp.zeros((_BLK, _N), jnp.float32)
    tot_acc, vio_acc = lax.fori_loop(0, _NBLK, body, (zeros, zeros))

    total = jnp.sum(tot_acc)
    viol = jnp.sum(vio_acc)

    ones_n = jnp.ones((1, _N), jnp.float32)
    p_row = lax.dot_general(ones_n, apf, (((1,), (1,)), ((), ())),
                            preferred_element_type=jnp.float32)
    m_row = lax.dot_general(ones_n, negf, (((1,), (1,)), ((), ())),
                            preferred_element_type=jnp.float32)
    count = jnp.sum(p_row * m_row)

    has = count > 0.5
    loss_sum = jnp.where(has, total, jnp.float32(1.0))
    ratio = jnp.where(has, viol / jnp.maximum(count, 1.0),
                      jnp.float32(1.0))
    loss_ref[...] = jnp.broadcast_to(loss_sum, (1, 1))
    ratio_ref[...] = jnp.broadcast_to(ratio, (1, 1))


def kernel(embeddings, target):
    t32 = target.astype(jnp.int32)
    perm = jnp.argsort(t32)
    ts = t32[perm]
    es = embeddings.astype(jnp.float32)[perm]
    i0s = _BLK * jnp.arange(_NBLK, dtype=jnp.int32)
    thr = ts[i0s + _BLK - 1]
    ends = jnp.sum(ts[None, :] <= thr[:, None], axis=1).astype(jnp.int32)
    kx = jnp.maximum(0, -((ends - i0s - _W) // -_W)).astype(jnp.int32)
    loss, ratio = pl.pallas_call(
        _triplet_kernel,
        out_shape=(jax.ShapeDtypeStruct((1, 1), jnp.float32),
                   jax.ShapeDtypeStruct((1, 1), jnp.float32)),
        in_specs=[pl.BlockSpec(memory_space=pltpu.VMEM),
                  pl.BlockSpec(memory_space=pltpu.VMEM),
                  pl.BlockSpec(memory_space=pltpu.VMEM),
                  pl.BlockSpec(memory_space=pltpu.SMEM)],
        scratch_shapes=[pltpu.VMEM((_N, _N), jnp.float32),
                        pltpu.VMEM((_NBLK, _PAD, _BLK), jnp.float32)],
    )(es, ts.reshape(1, _N), ts.reshape(_N, 1), kx.reshape(1, _NBLK))
    return (loss[0, 0], ratio[0, 0])
